# baseline (speedup 1.0000x reference)
import functools

import jax
import jax.numpy as jnp
from jax import lax
from jax.experimental import pallas as pl
from jax.experimental.pallas import tpu as pltpu
from jax.experimental.pallas import tpu_sc as plsc

D_MODEL = 2048
D_FF = 4096
E = 8
TOP_K = 2
TOKENS = 4096
REP = TOKENS * TOP_K
CAP = REP // E
DUMP = REP
XS_ROWS = REP + 8
LANES = 128

T_BLK = 512
A_BLK = T_BLK * TOP_K
N_STEPS = TOKENS // T_BLK


def _router_body(x_ref, rw_ref, rb_ref, slot_ref, gslot_ref, gate_ref,
                 aux_ref, base_ref, imp_ref, load_ref):
    step = pl.program_id(0)

    @pl.when(step == 0)
    def _init():
        base_ref[...] = jnp.zeros_like(base_ref)
        imp_ref[...] = jnp.zeros_like(imp_ref)
        load_ref[...] = jnp.zeros_like(load_ref)

    xb = x_ref[...]
    logits = jax.lax.dot_general(
        xb, rw_ref[...], (((1,), (1,)), ((), ())),
        preferred_element_type=jnp.float32) + rb_ref[...]
    m = jnp.max(logits, axis=1, keepdims=True)
    ex = jnp.exp(logits - m)
    probs = ex / jnp.sum(ex, axis=1, keepdims=True)

    lane = jax.lax.broadcasted_iota(
        jnp.int32, (T_BLK, LANES), 1).astype(jnp.float32)
    p1 = jnp.max(probs, axis=1, keepdims=True)
    i1 = jnp.min(jnp.where(probs == p1, lane, float(LANES)), axis=1,
                 keepdims=True)
    oh1 = (lane == i1).astype(jnp.float32)
    probs_m = jnp.where(oh1 > 0, -1.0, probs)
    p2 = jnp.max(probs_m, axis=1, keepdims=True)
    i2 = jnp.min(jnp.where(probs_m == p2, lane, float(LANES)), axis=1,
                 keepdims=True)
    oh2 = (lane == i2).astype(jnp.float32)

    A = jnp.stack([oh1, oh2], axis=1).reshape(A_BLK, LANES)
    probs_rep = jnp.stack([probs, probs], axis=1).reshape(A_BLK, LANES)
    ssum = jnp.clip(p1 + p2, 1e-9, None)
    ssum_rep = jnp.stack([ssum, ssum], axis=1).reshape(A_BLK, 1)
    p_j = jnp.sum(probs_rep * A, axis=1, keepdims=True)
    g_j = p_j / ssum_rep

    r_io = jax.lax.broadcasted_iota(jnp.int32, (A_BLK, A_BLK), 0)
    c_io = jax.lax.broadcasted_iota(jnp.int32, (A_BLK, A_BLK), 1)
    tri = (c_io < r_io).astype(jnp.float32)
    excl = jax.lax.dot_general(
        tri, A, (((1,), (0,)), ((), ())), preferred_element_type=jnp.float32)
    rank = (jnp.sum(excl * A, axis=1, keepdims=True)
            + jnp.sum(base_ref[...] * A, axis=1, keepdims=True))
    lane_a = jax.lax.broadcasted_iota(
        jnp.int32, (A_BLK, LANES), 1).astype(jnp.float32)
    e_j = jnp.sum(lane_a * A, axis=1, keepdims=True)

    valid = rank < float(CAP)
    slot_f = e_j * float(CAP) + jnp.minimum(rank, float(CAP - 1))
    slot_i = slot_f.astype(jnp.int32)
    slot_ref[...] = jnp.where(valid, slot_i, DUMP)
    gslot_ref[...] = jnp.where(valid, slot_i, 0)
    gate_eff = jnp.where(valid, g_j, 0.0)
    gate_ref[...] = jnp.broadcast_to(gate_eff, (A_BLK, 16))

    base_ref[...] += jnp.sum(A, axis=0, keepdims=True)
    imp_ref[...] += jnp.sum(probs, axis=0, keepdims=True)
    load_ref[...] += jnp.sum(oh1, axis=0, keepdims=True)

    aux = 0.01 * float(E) * jnp.sum(imp_ref[...] * load_ref[...]) / (
        float(TOKENS) * float(TOKENS))
    aux_ref[...] = jnp.full((1, LANES), aux, jnp.float32)


def _run_router(x_flat, router_w, router_b):
    rwp = jnp.zeros((LANES, D_MODEL), jnp.float32).at[:E].set(router_w)
    rbp = jnp.full((1, LANES), -1e30, jnp.float32).at[0, :E].set(router_b)
    slots, gslots, gates, aux = pl.pallas_call(
        _router_body,
        grid=(N_STEPS,),
        in_specs=[
            pl.BlockSpec((T_BLK, D_MODEL), lambda i: (i, 0)),
            pl.BlockSpec((LANES, D_MODEL), lambda i: (0, 0)),
            pl.BlockSpec((1, LANES), lambda i: (0, 0)),
        ],
        out_specs=[
            pl.BlockSpec((A_BLK, 1), lambda i: (i, 0)),
            pl.BlockSpec((A_BLK, 1), lambda i: (i, 0)),
            pl.BlockSpec((A_BLK, 16), lambda i: (i, 0)),
            pl.BlockSpec((1, LANES), lambda i: (0, 0)),
        ],
        out_shape=[
            jax.ShapeDtypeStruct((REP, 1), jnp.int32),
            jax.ShapeDtypeStruct((REP, 1), jnp.int32),
            jax.ShapeDtypeStruct((REP, 16), jnp.float32),
            jax.ShapeDtypeStruct((1, LANES), jnp.float32),
        ],
        scratch_shapes=[
            pltpu.VMEM((1, LANES), jnp.float32),
            pltpu.VMEM((1, LANES), jnp.float32),
            pltpu.VMEM((1, LANES), jnp.float32),
        ],
    )(x_flat, rwp, rbp)
    return slots.reshape(REP), gslots.reshape(REP), gates, aux[0, 0]


FF_BLK = 512
N_FF = D_FF // FF_BLK


def _cast_body(xs_ref, xb_ref):
    xb = xs_ref[...]
    xb_ref[...] = jnp.where(
        xb != xb, 0.0, jnp.clip(xb, -1e4, 1e4)).astype(jnp.bfloat16)


def _run_cast(xs):
    return pl.pallas_call(
        _cast_body,
        grid=(E,),
        in_specs=[pl.BlockSpec((CAP, D_MODEL), lambda e: (e, 0))],
        out_specs=pl.BlockSpec((CAP, D_MODEL), lambda e: (e, 0)),
        out_shape=jax.ShapeDtypeStruct((REP, D_MODEL), jnp.bfloat16),
    )(xs)


def _ffn_body(xs_ref, w1_ref, w3_ref, w2_ref, ys_ref):
    f = pl.program_id(1)
    xb = xs_ref[...]
    h1 = jax.lax.dot_general(
        xb, w1_ref[0].astype(jnp.bfloat16), (((1,), (1,)), ((), ())),
        preferred_element_type=jnp.float32)
    h3 = jax.lax.dot_general(
        xb, w3_ref[0].astype(jnp.bfloat16), (((1,), (1,)), ((), ())),
        preferred_element_type=jnp.float32)
    h = (h1 * jax.nn.sigmoid(h1) * h3).astype(jnp.bfloat16)
    y = jax.lax.dot_general(
        h, w2_ref[0].astype(jnp.bfloat16), (((1,), (1,)), ((), ())),
        preferred_element_type=jnp.float32)

    @pl.when(f == 0)
    def _first():
        ys_ref[...] = y

    @pl.when(f != 0)
    def _rest():
        ys_ref[...] += y


def _run_ffn(xs, w1, w3, w2):
    xsb = _run_cast(xs)
    return pl.pallas_call(
        _ffn_body,
        grid=(E, N_FF),
        in_specs=[
            pl.BlockSpec((CAP, D_MODEL), lambda e, f: (e, 0)),
            pl.BlockSpec((1, FF_BLK, D_MODEL), lambda e, f: (e, f, 0)),
            pl.BlockSpec((1, FF_BLK, D_MODEL), lambda e, f: (e, f, 0)),
            pl.BlockSpec((1, D_MODEL, FF_BLK), lambda e, f: (e, 0, f)),
        ],
        out_specs=pl.BlockSpec((CAP, D_MODEL), lambda e, f: (e, 0)),
        out_shape=jax.ShapeDtypeStruct((REP, D_MODEL), jnp.float32),
    )(xsb, w1, w3, w2)


NW = 32
A_PER_W = REP // NW
CH = 16
NCH = A_PER_W // CH


def _sc_dispatch_body(x_hbm, slot_hbm, xs_hbm, slot_m, tok_m,
                      rows0, rows1, gsem0, gsem1, ssem0, ssem1):
    wid = lax.axis_index("s") * 2 + lax.axis_index("c")
    a_base = wid * A_PER_W

    pltpu.sync_copy(slot_hbm.at[pl.ds(wid * NCH, NCH), :], slot_m)
    for c in range(NCH):
        ids = a_base + c * CH + lax.iota(jnp.int32, 16)
        tok_m[c, :] = lax.shift_right_logical(ids, 1)

    def gather(c, rows, sem):
        return pltpu.make_async_copy(x_hbm.at[tok_m.at[c]], rows, sem)

    def scatter(c, rows, sem):
        return pltpu.make_async_copy(rows, xs_hbm.at[slot_m.at[c]], sem)

    gather(0, rows0, gsem0).start()

    def pair(i, carry):
        c0 = 2 * i
        c1 = c0 + 1

        @pl.when(i > 0)
        def _():
            scatter(c1, rows1, ssem1).wait()

        gather(c1, rows1, gsem1).start()
        gather(c0, rows0, gsem0).wait()
        scatter(c0, rows0, ssem0).start()

        @pl.when(i < NCH // 2 - 1)
        def _():
            scatter(c0, rows0, ssem0).wait()
            gather(c0 + 2, rows0, gsem0).start()

        gather(c1, rows1, gsem1).wait()
        scatter(c1, rows1, ssem1).start()
        return carry

    lax.fori_loop(0, NCH // 2, pair, 0)
    scatter(NCH - 2, rows0, ssem0).wait()
    scatter(NCH - 1, rows1, ssem1).wait()


T_PER_W = TOKENS // NW
CT = 8
NCC = T_PER_W // CT
NGRP = D_MODEL // 16


def _sc_combine_body(ys_hbm, gslot_hbm, gate_hbm, out_hbm, slot_all,
                     gb0, gb1, rows0, rows1, out0, out1,
                     gsem0, gsem1, osem0, osem1):
    wid = lax.axis_index("s") * 2 + lax.axis_index("c")
    t_base = wid * T_PER_W
    a_base = 2 * t_base

    pltpu.sync_copy(gslot_hbm.at[pl.ds(a_base, A_PER_W)], slot_all)

    def fetch(c, rows, gb, sem):
        idx = slot_all.at[pl.ds(c * 2 * CT, 2 * CT)]
        g = pltpu.make_async_copy(ys_hbm.at[idx], rows, sem)
        b = pltpu.make_async_copy(
            gate_hbm.at[pl.ds(a_base + c * 2 * CT, 2 * CT), :], gb, sem)
        return g, b

    def owrite(c, out, sem):
        return pltpu.make_async_copy(
            out, out_hbm.at[pl.ds(t_base + c * CT, CT)], sem)

    def start(c, rows, gb, sem):
        g, b = fetch(c, rows, gb, sem)
        g.start()
        b.start()

    def drain(c, rows, gb, sem):
        g, b = fetch(c, rows, gb, sem)
        g.wait()
        b.wait()

    def compute(rows_v, gb_v, out_v):
        def grp(l, carry2):
            for u in range(4):
                off = l * 64 + u * 16
                for t in range(CT):
                    y0 = rows_v[2 * t, pl.ds(off, 16)]
                    y1 = rows_v[2 * t + 1, pl.ds(off, 16)]
                    g0 = gb_v[2 * t, :]
                    g1 = gb_v[2 * t + 1, :]
                    out_v[t, pl.ds(off, 16)] = g0 * y0 + g1 * y1
            return carry2

        lax.fori_loop(0, NGRP // 4, grp, 0)

    start(0, rows0, gb0, gsem0)

    def pair(i, carry):
        c0 = 2 * i
        c1 = c0 + 1
        start(c1, rows1, gb1, gsem1)
        drain(c0, rows0, gb0, gsem0)

        @pl.when(i > 0)
        def _():
            owrite(c0, out0, osem0).wait()

        compute(rows0, gb0, out0)
        owrite(c0, out0, osem0).start()

        @pl.when(i < NCC // 2 - 1)
        def _():
            start(c0 + 2, rows0, gb0, gsem0)

        drain(c1, rows1, gb1, gsem1)

        @pl.when(i > 0)
        def _():
            owrite(c1, out1, osem1).wait()

        compute(rows1, gb1, out1)
        owrite(c1, out1, osem1).start()
        return carry

    lax.fori_loop(0, NCC // 2, pair, 0)
    owrite(NCC - 2, out0, osem0).wait()
    owrite(NCC - 1, out1, osem1).wait()


@functools.lru_cache(maxsize=1)
def _get_sc_kernels():
    mesh = plsc.VectorSubcoreMesh(
        core_axis_name="c", subcore_axis_name="s",
        num_cores=2, num_subcores=16)
    dispatch = pl.kernel(
        _sc_dispatch_body,
        mesh=mesh,
        out_type=jax.ShapeDtypeStruct((XS_ROWS, D_MODEL), jnp.float32),
        scratch_types=[
            pltpu.VMEM((NCH, CH), jnp.int32),
            pltpu.VMEM((NCH, CH), jnp.int32),
            pltpu.VMEM((CH, D_MODEL), jnp.float32),
            pltpu.VMEM((CH, D_MODEL), jnp.float32),
            pltpu.SemaphoreType.DMA,
            pltpu.SemaphoreType.DMA,
            pltpu.SemaphoreType.DMA,
            pltpu.SemaphoreType.DMA,
        ],
    )
    combine = pl.kernel(
        _sc_combine_body,
        mesh=mesh,
        out_type=jax.ShapeDtypeStruct((TOKENS, D_MODEL), jnp.float32),
        scratch_types=[
            pltpu.VMEM((A_PER_W,), jnp.int32),
            pltpu.VMEM((2 * CT, 16), jnp.float32),
            pltpu.VMEM((2 * CT, 16), jnp.float32),
            pltpu.VMEM((2 * CT, D_MODEL), jnp.float32),
            pltpu.VMEM((2 * CT, D_MODEL), jnp.float32),
            pltpu.VMEM((CT, D_MODEL), jnp.float32),
            pltpu.VMEM((CT, D_MODEL), jnp.float32),
            pltpu.SemaphoreType.DMA,
            pltpu.SemaphoreType.DMA,
            pltpu.SemaphoreType.DMA,
            pltpu.SemaphoreType.DMA,
        ],
    )
    return dispatch, combine


def kernel(x, router_w, router_b, w1, w3, w2):
    b, s, d = x.shape
    sc_dispatch, sc_combine = _get_sc_kernels()
    x_flat = x.reshape(TOKENS, d)
    slots, gslots, gates, aux = _run_router(x_flat, router_w, router_b)
    xs = sc_dispatch(x_flat, slots.reshape(REP // CH, CH))
    ys = _run_ffn(xs, w1, w3, w2)
    out = sc_combine(ys, gslots, gates)
    return out.reshape(b, s, d), aux


# speedup vs baseline: 1.0168x; 1.0168x over previous
import functools

import jax
import jax.numpy as jnp
from jax import lax
from jax.experimental import pallas as pl
from jax.experimental.pallas import tpu as pltpu
from jax.experimental.pallas import tpu_sc as plsc

D_MODEL = 2048
D_FF = 4096
E = 8
TOP_K = 2
TOKENS = 4096
REP = TOKENS * TOP_K
CAP = REP // E
DUMP = REP
XS_ROWS = REP + 8
LANES = 128

T_BLK = 512
A_BLK = T_BLK * TOP_K
N_STEPS = TOKENS // T_BLK


def _router_body(x_ref, rw_ref, rb_ref, slot_ref, gslot_ref, gate_ref,
                 aux_ref, base_ref, imp_ref, load_ref):
    step = pl.program_id(0)

    @pl.when(step == 0)
    def _init():
        base_ref[...] = jnp.zeros_like(base_ref)
        imp_ref[...] = jnp.zeros_like(imp_ref)
        load_ref[...] = jnp.zeros_like(load_ref)

    xb = x_ref[...]
    logits = jax.lax.dot_general(
        xb, rw_ref[...], (((1,), (1,)), ((), ())),
        preferred_element_type=jnp.float32) + rb_ref[...]
    m = jnp.max(logits, axis=1, keepdims=True)
    ex = jnp.exp(logits - m)
    probs = ex / jnp.sum(ex, axis=1, keepdims=True)

    lane = jax.lax.broadcasted_iota(
        jnp.int32, (T_BLK, LANES), 1).astype(jnp.float32)
    p1 = jnp.max(probs, axis=1, keepdims=True)
    i1 = jnp.min(jnp.where(probs == p1, lane, float(LANES)), axis=1,
                 keepdims=True)
    oh1 = (lane == i1).astype(jnp.float32)
    probs_m = jnp.where(oh1 > 0, -1.0, probs)
    p2 = jnp.max(probs_m, axis=1, keepdims=True)
    i2 = jnp.min(jnp.where(probs_m == p2, lane, float(LANES)), axis=1,
                 keepdims=True)
    oh2 = (lane == i2).astype(jnp.float32)

    A = jnp.stack([oh1, oh2], axis=1).reshape(A_BLK, LANES)
    probs_rep = jnp.stack([probs, probs], axis=1).reshape(A_BLK, LANES)
    ssum = jnp.clip(p1 + p2, 1e-9, None)
    ssum_rep = jnp.stack([ssum, ssum], axis=1).reshape(A_BLK, 1)
    p_j = jnp.sum(probs_rep * A, axis=1, keepdims=True)
    g_j = p_j / ssum_rep

    r_io = jax.lax.broadcasted_iota(jnp.int32, (A_BLK, A_BLK), 0)
    c_io = jax.lax.broadcasted_iota(jnp.int32, (A_BLK, A_BLK), 1)
    tri = (c_io < r_io).astype(jnp.float32)
    excl = jax.lax.dot_general(
        tri, A, (((1,), (0,)), ((), ())), preferred_element_type=jnp.float32)
    rank = (jnp.sum(excl * A, axis=1, keepdims=True)
            + jnp.sum(base_ref[...] * A, axis=1, keepdims=True))
    lane_a = jax.lax.broadcasted_iota(
        jnp.int32, (A_BLK, LANES), 1).astype(jnp.float32)
    e_j = jnp.sum(lane_a * A, axis=1, keepdims=True)

    valid = rank < float(CAP)
    slot_f = e_j * float(CAP) + jnp.minimum(rank, float(CAP - 1))
    slot_i = slot_f.astype(jnp.int32)
    slot_ref[...] = jnp.where(valid, slot_i, DUMP)
    gslot_ref[...] = jnp.where(valid, slot_i, 0)
    gate_eff = jnp.where(valid, g_j, 0.0)
    gate_ref[...] = jnp.broadcast_to(gate_eff, (A_BLK, 16))

    base_ref[...] += jnp.sum(A, axis=0, keepdims=True)
    imp_ref[...] += jnp.sum(probs, axis=0, keepdims=True)
    load_ref[...] += jnp.sum(oh1, axis=0, keepdims=True)

    aux = 0.01 * float(E) * jnp.sum(imp_ref[...] * load_ref[...]) / (
        float(TOKENS) * float(TOKENS))
    aux_ref[...] = jnp.full((1, LANES), aux, jnp.float32)


def _run_router(x_flat, router_w, router_b):
    rwp = jnp.zeros((LANES, D_MODEL), jnp.float32).at[:E].set(router_w)
    rbp = jnp.full((1, LANES), -1e30, jnp.float32).at[0, :E].set(router_b)
    slots, gslots, gates, aux = pl.pallas_call(
        _router_body,
        grid=(N_STEPS,),
        in_specs=[
            pl.BlockSpec((T_BLK, D_MODEL), lambda i: (i, 0)),
            pl.BlockSpec((LANES, D_MODEL), lambda i: (0, 0)),
            pl.BlockSpec((1, LANES), lambda i: (0, 0)),
        ],
        out_specs=[
            pl.BlockSpec((A_BLK, 1), lambda i: (i, 0)),
            pl.BlockSpec((A_BLK, 1), lambda i: (i, 0)),
            pl.BlockSpec((A_BLK, 16), lambda i: (i, 0)),
            pl.BlockSpec((1, LANES), lambda i: (0, 0)),
        ],
        out_shape=[
            jax.ShapeDtypeStruct((REP, 1), jnp.int32),
            jax.ShapeDtypeStruct((REP, 1), jnp.int32),
            jax.ShapeDtypeStruct((REP, 16), jnp.float32),
            jax.ShapeDtypeStruct((1, LANES), jnp.float32),
        ],
        scratch_shapes=[
            pltpu.VMEM((1, LANES), jnp.float32),
            pltpu.VMEM((1, LANES), jnp.float32),
            pltpu.VMEM((1, LANES), jnp.float32),
        ],
    )(x_flat, rwp, rbp)
    return slots.reshape(REP), gslots.reshape(REP), gates, aux[0, 0]


FF_BLK = 512
N_FF = D_FF // FF_BLK


def _cast_body(xs_ref, xb_ref):
    xb = xs_ref[...]
    xb_ref[...] = jnp.where(
        xb != xb, 0.0, jnp.clip(xb, -1e4, 1e4)).astype(jnp.bfloat16)


def _run_cast(xs):
    return pl.pallas_call(
        _cast_body,
        grid=(E,),
        in_specs=[pl.BlockSpec((CAP, D_MODEL), lambda e: (e, 0))],
        out_specs=pl.BlockSpec((CAP, D_MODEL), lambda e: (e, 0)),
        out_shape=jax.ShapeDtypeStruct((REP, D_MODEL), jnp.bfloat16),
    )(xs)


def _ffn_body(xs_ref, w1_ref, w3_ref, w2_ref, ys_ref):
    f = pl.program_id(1)
    xb = xs_ref[...]
    h1 = jax.lax.dot_general(
        xb, w1_ref[0].astype(jnp.bfloat16), (((1,), (1,)), ((), ())),
        preferred_element_type=jnp.float32)
    h3 = jax.lax.dot_general(
        xb, w3_ref[0].astype(jnp.bfloat16), (((1,), (1,)), ((), ())),
        preferred_element_type=jnp.float32)
    h = (h1 * jax.nn.sigmoid(h1) * h3).astype(jnp.bfloat16)
    y = jax.lax.dot_general(
        h, w2_ref[0].astype(jnp.bfloat16), (((1,), (1,)), ((), ())),
        preferred_element_type=jnp.float32)

    @pl.when(f == 0)
    def _first():
        ys_ref[...] = y

    @pl.when(f != 0)
    def _rest():
        ys_ref[...] += y


def _run_ffn(xs, w1, w3, w2):
    xsb = _run_cast(xs)
    return pl.pallas_call(
        _ffn_body,
        grid=(E, N_FF),
        in_specs=[
            pl.BlockSpec((CAP, D_MODEL), lambda e, f: (e, 0)),
            pl.BlockSpec((1, FF_BLK, D_MODEL), lambda e, f: (e, f, 0)),
            pl.BlockSpec((1, FF_BLK, D_MODEL), lambda e, f: (e, f, 0)),
            pl.BlockSpec((1, D_MODEL, FF_BLK), lambda e, f: (e, 0, f)),
        ],
        out_specs=pl.BlockSpec((CAP, D_MODEL), lambda e, f: (e, 0)),
        out_shape=jax.ShapeDtypeStruct((REP, D_MODEL), jnp.float32),
    )(xsb, w1, w3, w2)


NW = 32
A_PER_W = REP // NW
CH = 16
NCH = A_PER_W // CH


def _sc_dispatch_body(x_hbm, slot_hbm, xs_hbm, slot_m, tok_m,
                      rows0, rows1, gsem0, gsem1, ssem0, ssem1):
    wid = lax.axis_index("s") * 2 + lax.axis_index("c")
    a_base = wid * A_PER_W

    pltpu.sync_copy(slot_hbm.at[pl.ds(wid * NCH, NCH), :], slot_m)
    for c in range(NCH):
        ids = a_base + c * CH + lax.iota(jnp.int32, 16)
        tok_m[c, :] = lax.shift_right_logical(ids, 1)

    def gather(c, rows, sem):
        return pltpu.make_async_copy(x_hbm.at[tok_m.at[c]], rows, sem)

    def scatter(c, rows, sem):
        return pltpu.make_async_copy(rows, xs_hbm.at[slot_m.at[c]], sem)

    gather(0, rows0, gsem0).start()

    def pair(i, carry):
        c0 = 2 * i
        c1 = c0 + 1

        @pl.when(i > 0)
        def _():
            scatter(c1, rows1, ssem1).wait()

        gather(c1, rows1, gsem1).start()
        gather(c0, rows0, gsem0).wait()
        scatter(c0, rows0, ssem0).start()

        @pl.when(i < NCH // 2 - 1)
        def _():
            scatter(c0, rows0, ssem0).wait()
            gather(c0 + 2, rows0, gsem0).start()

        gather(c1, rows1, gsem1).wait()
        scatter(c1, rows1, ssem1).start()
        return carry

    lax.fori_loop(0, NCH // 2, pair, 0)
    scatter(NCH - 2, rows0, ssem0).wait()
    scatter(NCH - 1, rows1, ssem1).wait()


T_PER_W = TOKENS // NW
CT = 8
NCC = T_PER_W // CT
NGRP = D_MODEL // 16


def _sc_combine_body(ys_hbm, gslot_hbm, gate_hbm, out_hbm, slot_all,
                     gb0, gb1, rows0, rows1, out0, out1,
                     gsem0, gsem1, osem0, osem1):
    wid = lax.axis_index("s") * 2 + lax.axis_index("c")
    t_base = wid * T_PER_W
    a_base = 2 * t_base

    pltpu.sync_copy(gslot_hbm.at[pl.ds(a_base, A_PER_W)], slot_all)

    def fetch(c, rows, gb, sem):
        idx = slot_all.at[pl.ds(c * 2 * CT, 2 * CT)]
        g = pltpu.make_async_copy(ys_hbm.at[idx], rows, sem)
        b = pltpu.make_async_copy(
            gate_hbm.at[pl.ds(a_base + c * 2 * CT, 2 * CT), :], gb, sem)
        return g, b

    def owrite(c, out, sem):
        return pltpu.make_async_copy(
            out, out_hbm.at[pl.ds(t_base + c * CT, CT)], sem)

    def start(c, rows, gb, sem):
        g, b = fetch(c, rows, gb, sem)
        g.start()
        b.start()

    def drain(c, rows, gb, sem):
        g, b = fetch(c, rows, gb, sem)
        g.wait()
        b.wait()

    def compute(rows_v, gb_v, out_v):
        def grp(l, carry2):
            off = l * 16
            for t in range(CT):
                y0 = rows_v[2 * t, pl.ds(off, 16)]
                y1 = rows_v[2 * t + 1, pl.ds(off, 16)]
                g0 = gb_v[2 * t, :]
                g1 = gb_v[2 * t + 1, :]
                out_v[t, pl.ds(off, 16)] = g0 * y0 + g1 * y1
            return carry2

        lax.fori_loop(0, NGRP, grp, 0)

    start(0, rows0, gb0, gsem0)

    def pair(i, carry):
        c0 = 2 * i
        c1 = c0 + 1
        start(c1, rows1, gb1, gsem1)
        drain(c0, rows0, gb0, gsem0)

        @pl.when(i > 0)
        def _():
            owrite(c0, out0, osem0).wait()

        compute(rows0, gb0, out0)
        owrite(c0, out0, osem0).start()

        @pl.when(i < NCC // 2 - 1)
        def _():
            start(c0 + 2, rows0, gb0, gsem0)

        drain(c1, rows1, gb1, gsem1)

        @pl.when(i > 0)
        def _():
            owrite(c1, out1, osem1).wait()

        compute(rows1, gb1, out1)
        owrite(c1, out1, osem1).start()
        return carry

    lax.fori_loop(0, NCC // 2, pair, 0)
    owrite(NCC - 2, out0, osem0).wait()
    owrite(NCC - 1, out1, osem1).wait()


@functools.lru_cache(maxsize=1)
def _get_sc_kernels():
    mesh = plsc.VectorSubcoreMesh(
        core_axis_name="c", subcore_axis_name="s",
        num_cores=2, num_subcores=16)
    dispatch = pl.kernel(
        _sc_dispatch_body,
        mesh=mesh,
        out_type=jax.ShapeDtypeStruct((XS_ROWS, D_MODEL), jnp.float32),
        scratch_types=[
            pltpu.VMEM((NCH, CH), jnp.int32),
            pltpu.VMEM((NCH, CH), jnp.int32),
            pltpu.VMEM((CH, D_MODEL), jnp.float32),
            pltpu.VMEM((CH, D_MODEL), jnp.float32),
            pltpu.SemaphoreType.DMA,
            pltpu.SemaphoreType.DMA,
            pltpu.SemaphoreType.DMA,
            pltpu.SemaphoreType.DMA,
        ],
    )
    combine = pl.kernel(
        _sc_combine_body,
        mesh=mesh,
        out_type=jax.ShapeDtypeStruct((TOKENS, D_MODEL), jnp.float32),
        scratch_types=[
            pltpu.VMEM((A_PER_W,), jnp.int32),
            pltpu.VMEM((2 * CT, 16), jnp.float32),
            pltpu.VMEM((2 * CT, 16), jnp.float32),
            pltpu.VMEM((2 * CT, D_MODEL), jnp.float32),
            pltpu.VMEM((2 * CT, D_MODEL), jnp.float32),
            pltpu.VMEM((CT, D_MODEL), jnp.float32),
            pltpu.VMEM((CT, D_MODEL), jnp.float32),
            pltpu.SemaphoreType.DMA,
            pltpu.SemaphoreType.DMA,
            pltpu.SemaphoreType.DMA,
            pltpu.SemaphoreType.DMA,
        ],
    )
    return dispatch, combine


def kernel(x, router_w, router_b, w1, w3, w2):
    b, s, d = x.shape
    sc_dispatch, sc_combine = _get_sc_kernels()
    x_flat = x.reshape(TOKENS, d)
    slots, gslots, gates, aux = _run_router(x_flat, router_w, router_b)
    xs = sc_dispatch(x_flat, slots.reshape(REP // CH, CH))
    ys = _run_ffn(xs, w1, w3, w2)
    out = sc_combine(ys, gslots, gates)
    return out.reshape(b, s, d), aux
